# baseline (speedup 1.0000x reference)
import functools

import jax
import jax.numpy as jnp
from jax import lax
from jax.experimental import pallas as pl
from jax.experimental.pallas import tpu as pltpu
from jax.experimental.pallas import tpu_sc as plsc

GS = 128
D = 256
H2 = 1024
NOUT = 512
NRES = 640
GB = 16
T = 4
GPW = 4


def _phase1_body(nn_ref, adj_ref, nodes_ref, obs_ref, wcat_ref, brel_ref,
                 g_ref, r_ref):
    blk = pl.program_id(0)
    rowi = lax.broadcasted_iota(jnp.int32, (GS, GS), 0)
    coli = lax.broadcasted_iota(jnp.int32, (GS, GS), 1)
    rvec = lax.broadcasted_iota(jnp.int32, (GS, 1), 0)
    for i in range(GB):
        n = nn_ref[blk * GB + i]
        A = (adj_ref[i] != 0).astype(jnp.bfloat16)
        sel = (rowi == n) & (coli == n)
        sel = sel | ((rowi == n) & (coli == n - 1) & (n > 0))
        sel = sel | ((rowi == n - 1) & (coli == n) & (n > 0))
        A = jnp.where(sel, jnp.bfloat16(1.0), A)
        rmask = rvec == n
        X = jnp.where(rmask,
                      jnp.broadcast_to(obs_ref[i, 0, :][None, :], (GS, D)),
                      nodes_ref[i]).astype(jnp.bfloat16)
        agg0 = lax.dot_general(A, X, (((0,), (0,)), ((), ())),
                               preferred_element_type=jnp.float32)
        Z = jnp.concatenate([agg0.astype(jnp.bfloat16), X], axis=1)
        h1 = lax.dot_general(Z, wcat_ref[...], (((1,), (0,)), ((), ())),
                             preferred_element_type=jnp.float32)
        h1 = jnp.maximum(h1 + brel_ref[0, :][None, :], 0.0)
        wcol = jnp.where(coli == n, A, jnp.bfloat16(0.0)).sum(axis=1)
        en = (rvec[:, 0] == n).astype(jnp.float32)
        sel2 = jnp.concatenate([wcol[None, :].astype(jnp.float32),
                                en[None, :]], axis=0)
        GR = lax.dot_general(sel2, h1, (((1,), (0,)), ((), ())),
                             preferred_element_type=jnp.float32)
        g_ref[i, 0, :] = GR[0]
        r_ref[i, 0, :] = GR[1]


def _phase2_body(g_ref, r_ref, wrel_ref, wroot_ref, brel_ref, whead_ref,
                 bhead_ref, res_ref):
    h2 = (g_ref[...] @ wrel_ref[...] + r_ref[...] @ wroot_ref[...]
          + brel_ref[...])
    h2 = jnp.maximum(h2, 0.0)
    res_ref[...] = h2 @ whead_ref[...] + bhead_ref[...]


def _sc_scatter_body(res_hbm, lg_hbm, vl_hbm, resbuf_v, block_v, vals_v):
    wid = lax.axis_index("s") * 2 + lax.axis_index("c")
    half = wid % 2
    zero16 = jnp.zeros((16,), jnp.float32)
    for r in range(T * GPW):
        if r % T != 0:
            for c in range(0, NOUT, 16):
                block_v[0, r, pl.ds(c, 16)] = zero16
    pltpu.sync_copy(res_hbm.at[pl.ds(wid // 2, 1)], resbuf_v)
    for b in range(GPW):
        for c in range(0, NOUT, 16):
            block_v[0, b * T, pl.ds(c, 16)] = (
                resbuf_v[0, half * GPW + b, pl.ds(c, 16)])
    lanes = lax.broadcasted_iota(jnp.int32, (16,), 0)
    acc = jnp.zeros((16,), jnp.float32)
    for b in range(GPW):
        v = resbuf_v[0, half * GPW + b, pl.ds(NOUT, 16)]
        acc = jnp.where(lanes == b * T, jnp.broadcast_to(v[0], (16,)), acc)
    vals_v[...] = acc
    pltpu.sync_copy(block_v, lg_hbm.at[pl.ds(wid, 1)])
    pltpu.sync_copy(vals_v, vl_hbm.at[pl.ds(wid * 16, 16)])


def kernel(obs_flat, seq_lens, num_nodes, nodes, adj_mats, W_rel0, b_rel0,
           W_root0, W_rel1, b_rel1, W_root1, W_logit, b_logit, W_value,
           b_value):
    B = seq_lens.shape[0]
    obs0 = obs_flat.reshape(B, T, D)[:, 0, :]
    nn = num_nodes.reshape(B).astype(jnp.int32)
    w_cat0 = jnp.concatenate([W_rel0.T, W_root0.T],
                             axis=0).astype(jnp.bfloat16)

    grid_spec = pltpu.PrefetchScalarGridSpec(
        num_scalar_prefetch=1,
        grid=(B // GB,),
        in_specs=[
            pl.BlockSpec((GB, GS, GS), lambda b, nn_: (b, 0, 0)),
            pl.BlockSpec((GB, GS, D), lambda b, nn_: (b, 0, 0)),
            pl.BlockSpec((GB, 1, D), lambda b, nn_: (b, 0, 0)),
            pl.BlockSpec((2 * D, D), lambda b, nn_: (0, 0)),
            pl.BlockSpec((1, D), lambda b, nn_: (0, 0)),
        ],
        out_specs=[
            pl.BlockSpec((GB, 1, D), lambda b, nn_: (b, 0, 0)),
            pl.BlockSpec((GB, 1, D), lambda b, nn_: (b, 0, 0)),
        ],
    )
    G, R = pl.pallas_call(
        _phase1_body,
        grid_spec=grid_spec,
        out_shape=[
            jax.ShapeDtypeStruct((B, 1, D), jnp.float32),
            jax.ShapeDtypeStruct((B, 1, D), jnp.float32),
        ],
    )(nn, adj_mats, nodes, obs0.reshape(B, 1, D), w_cat0,
      b_rel0.reshape(1, D))
    G = G.reshape(B, D)
    R = R.reshape(B, D)

    w_head = jnp.concatenate(
        [W_logit.T, W_value.T, jnp.zeros((H2, NRES - NOUT - 1), jnp.float32)],
        axis=1)
    b_head = jnp.concatenate(
        [b_logit, b_value, jnp.zeros((NRES - NOUT - 1,), jnp.float32)])
    res = pl.pallas_call(
        _phase2_body,
        out_shape=jax.ShapeDtypeStruct((B, NRES), jnp.float32),
    )(G, R, W_rel1.T, W_root1.T, b_rel1.reshape(1, H2), w_head,
      b_head.reshape(1, NRES))

    mesh = plsc.VectorSubcoreMesh(core_axis_name="c", subcore_axis_name="s")
    sc_scatter = functools.partial(
        pl.kernel, mesh=mesh,
        out_type=[
            jax.ShapeDtypeStruct((32, T * GPW, NOUT), jnp.float32),
            jax.ShapeDtypeStruct((B * T,), jnp.float32),
        ],
        scratch_types=[
            pltpu.VMEM((1, 2 * GPW, NRES), jnp.float32),
            pltpu.VMEM((1, T * GPW, NOUT), jnp.float32),
            pltpu.VMEM((16,), jnp.float32),
        ],
    )(_sc_scatter_body)
    lg3, values = sc_scatter(res.reshape(16, 2 * GPW, NRES))
    return (lg3.reshape(B * T, NOUT), values)


# speedup vs baseline: 1.4969x; 1.4969x over previous
import functools

import jax
import jax.numpy as jnp
from jax import lax
from jax.experimental import pallas as pl
from jax.experimental.pallas import tpu as pltpu
from jax.experimental.pallas import tpu_sc as plsc

GS = 128
D = 256
H2 = 1024
NOUT = 512
NRES = 640
GB = 16
T = 4
GPW = 4


def _phase1_body(nn_ref, adj_ref, nodes_ref, obs_ref, wcat_ref, brel_ref,
                 g_ref, r_ref, z_ref, wc_ref):
    blk = pl.program_id(0)
    rowi = lax.broadcasted_iota(jnp.int32, (GS, GS), 0)
    coli = lax.broadcasted_iota(jnp.int32, (GS, GS), 1)
    rvec = lax.broadcasted_iota(jnp.int32, (GS, 1), 0)
    for i in range(GB):
        n = nn_ref[blk * GB + i]
        A = (adj_ref[i] != 0).astype(jnp.bfloat16)
        sel = (rowi == n) & (coli == n)
        sel = sel | ((rowi == n) & (coli == n - 1) & (n > 0))
        sel = sel | ((rowi == n - 1) & (coli == n) & (n > 0))
        A = jnp.where(sel, jnp.bfloat16(1.0), A)
        rmask = rvec == n
        X = jnp.where(rmask,
                      jnp.broadcast_to(obs_ref[i, 0, :][None, :], (GS, D)),
                      nodes_ref[i]).astype(jnp.bfloat16)
        agg0 = lax.dot_general(A, X, (((0,), (0,)), ((), ())),
                               preferred_element_type=jnp.float32)
        z_ref[pl.ds(i * GS, GS), :] = jnp.concatenate(
            [agg0.astype(jnp.bfloat16), X], axis=1)
        wcol = jnp.where(coli == n, A, jnp.bfloat16(0.0)).sum(axis=1)
        en = (rvec[:, 0] == n).astype(jnp.float32)
        wc_ref[2 * i, :] = wcol.astype(jnp.float32)
        wc_ref[2 * i + 1, :] = en
    h1 = lax.dot_general(z_ref[...], wcat_ref[...], (((1,), (0,)), ((), ())),
                         preferred_element_type=jnp.float32)
    h1 = jnp.maximum(h1 + brel_ref[0, :][None, :], 0.0)
    for i in range(GB):
        GR = lax.dot_general(wc_ref[pl.ds(2 * i, 2), :],
                             h1[i * GS:(i + 1) * GS],
                             (((1,), (0,)), ((), ())),
                             preferred_element_type=jnp.float32)
        g_ref[i, 0, :] = GR[0]
        r_ref[i, 0, :] = GR[1]


def _phase2_body(g_ref, r_ref, wrel_ref, wroot_ref, brel_ref, whead_ref,
                 bhead_ref, res_ref):
    h2 = (g_ref[...] @ wrel_ref[...] + r_ref[...] @ wroot_ref[...]
          + brel_ref[...])
    h2 = jnp.maximum(h2, 0.0)
    res_ref[...] = h2 @ whead_ref[...] + bhead_ref[...]


def _sc_scatter_body(res_hbm, lg_hbm, vl_hbm, resbuf_v, block_v, vals_v):
    wid = lax.axis_index("s") * 2 + lax.axis_index("c")
    half = wid % 2
    zero16 = jnp.zeros((16,), jnp.float32)
    for r in range(T * GPW):
        if r % T != 0:
            for c in range(0, NOUT, 16):
                block_v[0, r, pl.ds(c, 16)] = zero16
    pltpu.sync_copy(res_hbm.at[pl.ds(wid // 2, 1)], resbuf_v)
    for b in range(GPW):
        for c in range(0, NOUT, 16):
            block_v[0, b * T, pl.ds(c, 16)] = (
                resbuf_v[0, half * GPW + b, pl.ds(c, 16)])
    lanes = lax.broadcasted_iota(jnp.int32, (16,), 0)
    acc = jnp.zeros((16,), jnp.float32)
    for b in range(GPW):
        v = resbuf_v[0, half * GPW + b, pl.ds(NOUT, 16)]
        acc = jnp.where(lanes == b * T, jnp.broadcast_to(v[0], (16,)), acc)
    vals_v[...] = acc
    pltpu.sync_copy(block_v, lg_hbm.at[pl.ds(wid, 1)])
    pltpu.sync_copy(vals_v, vl_hbm.at[pl.ds(wid * 16, 16)])


def kernel(obs_flat, seq_lens, num_nodes, nodes, adj_mats, W_rel0, b_rel0,
           W_root0, W_rel1, b_rel1, W_root1, W_logit, b_logit, W_value,
           b_value):
    B = seq_lens.shape[0]
    obs0 = obs_flat.reshape(B, T, D)[:, 0, :]
    nn = num_nodes.reshape(B).astype(jnp.int32)
    w_cat0 = jnp.concatenate([W_rel0.T, W_root0.T],
                             axis=0).astype(jnp.bfloat16)

    grid_spec = pltpu.PrefetchScalarGridSpec(
        num_scalar_prefetch=1,
        grid=(B // GB,),
        in_specs=[
            pl.BlockSpec((GB, GS, GS), lambda b, nn_: (b, 0, 0)),
            pl.BlockSpec((GB, GS, D), lambda b, nn_: (b, 0, 0)),
            pl.BlockSpec((GB, 1, D), lambda b, nn_: (b, 0, 0)),
            pl.BlockSpec((2 * D, D), lambda b, nn_: (0, 0)),
            pl.BlockSpec((1, D), lambda b, nn_: (0, 0)),
        ],
        out_specs=[
            pl.BlockSpec((GB, 1, D), lambda b, nn_: (b, 0, 0)),
            pl.BlockSpec((GB, 1, D), lambda b, nn_: (b, 0, 0)),
        ],
        scratch_shapes=[
            pltpu.VMEM((GB * GS, 2 * D), jnp.bfloat16),
            pltpu.VMEM((2 * GB, GS), jnp.float32),
        ],
    )
    G, R = pl.pallas_call(
        _phase1_body,
        grid_spec=grid_spec,
        out_shape=[
            jax.ShapeDtypeStruct((B, 1, D), jnp.float32),
            jax.ShapeDtypeStruct((B, 1, D), jnp.float32),
        ],
    )(nn, adj_mats, nodes, obs0.reshape(B, 1, D), w_cat0,
      b_rel0.reshape(1, D))
    G = G.reshape(B, D)
    R = R.reshape(B, D)

    w_head = jnp.concatenate(
        [W_logit.T, W_value.T, jnp.zeros((H2, NRES - NOUT - 1), jnp.float32)],
        axis=1)
    b_head = jnp.concatenate(
        [b_logit, b_value, jnp.zeros((NRES - NOUT - 1,), jnp.float32)])
    res = pl.pallas_call(
        _phase2_body,
        out_shape=jax.ShapeDtypeStruct((B, NRES), jnp.float32),
    )(G, R, W_rel1.T, W_root1.T, b_rel1.reshape(1, H2), w_head,
      b_head.reshape(1, NRES))

    mesh = plsc.VectorSubcoreMesh(core_axis_name="c", subcore_axis_name="s")
    sc_scatter = functools.partial(
        pl.kernel, mesh=mesh,
        out_type=[
            jax.ShapeDtypeStruct((32, T * GPW, NOUT), jnp.float32),
            jax.ShapeDtypeStruct((B * T,), jnp.float32),
        ],
        scratch_types=[
            pltpu.VMEM((1, 2 * GPW, NRES), jnp.float32),
            pltpu.VMEM((1, T * GPW, NOUT), jnp.float32),
            pltpu.VMEM((16,), jnp.float32),
        ],
    )(_sc_scatter_body)
    lg3, values = sc_scatter(res.reshape(16, 2 * GPW, NRES))
    return (lg3.reshape(B * T, NOUT), values)
